# baseline (speedup 1.0000x reference)
import functools

import jax
import jax.numpy as jnp
from jax.experimental import pallas as pl

NEIGHBOUR_IOU = 0.2

_ROW_TILE = 200


def _iou_rows_kernel(dt_ref, gtc_ref, dtc_ref, dtgt_ref, dtdt_ref, mask_ref):
    d = dt_ref[...]
    x1r = d[:, 0:1]
    y1r = d[:, 1:2]
    x2r = d[:, 2:3]
    y2r = d[:, 3:4]
    ar = (x2r - x1r) * (y2r - y1r)

    def tile_iou(c):
        ix1 = jnp.maximum(x1r, c[0:1, :])
        iy1 = jnp.maximum(y1r, c[1:2, :])
        ix2 = jnp.minimum(x2r, c[2:3, :])
        iy2 = jnp.minimum(y2r, c[3:4, :])
        inter = jnp.maximum(ix2 - ix1, 0.0) * jnp.maximum(iy2 - iy1, 0.0)
        union = ar + c[4:5, :] - inter
        return inter / union

    dtgt_ref[...] = tile_iou(gtc_ref[...])
    dd = tile_iou(dtc_ref[...])
    dtdt_ref[...] = dd
    mask_ref[...] = dd >= NEIGHBOUR_IOU


@functools.partial(jax.jit, static_argnames=())
def kernel(detections, gt_boxes):
    dt = detections[:2000]
    n_dt = dt.shape[0]
    n_gt = gt_boxes.shape[0]

    def cols(b):
        x1, y1, x2, y2 = b[:, 0], b[:, 1], b[:, 2], b[:, 3]
        area = (x2 - x1) * (y2 - y1)
        z = jnp.zeros_like(x1)
        return jnp.stack([x1, y1, x2, y2, area, z, z, z], axis=0)

    gtc = cols(gt_boxes)
    dtc = cols(dt)

    br = _ROW_TILE
    grid = (n_dt // br,)
    out = pl.pallas_call(
        _iou_rows_kernel,
        grid=grid,
        in_specs=[
            pl.BlockSpec((br, 4), lambda i: (i, 0)),
            pl.BlockSpec((8, n_gt), lambda i: (0, 0)),
            pl.BlockSpec((8, n_dt), lambda i: (0, 0)),
        ],
        out_specs=[
            pl.BlockSpec((br, n_gt), lambda i: (i, 0)),
            pl.BlockSpec((br, n_dt), lambda i: (i, 0)),
            pl.BlockSpec((br, n_dt), lambda i: (i, 0)),
        ],
        out_shape=[
            jax.ShapeDtypeStruct((n_dt, n_gt), jnp.float32),
            jax.ShapeDtypeStruct((n_dt, n_dt), jnp.float32),
            jax.ShapeDtypeStruct((n_dt, n_dt), jnp.bool_),
        ],
    )(dt, gtc, dtc)
    return out[0], out[1], out[2]
